# baseline (speedup 1.0000x reference)
import functools

import jax
import jax.numpy as jnp
from jax import lax
from jax.experimental import pallas as pl
from jax.experimental.pallas import tpu as pltpu
from jax.experimental.pallas import tpu_sc as plsc

ROWS = 4
COLS = 2048
EMB_DIM = 768
NUM_WORKERS = 32
W_PER_ROW = NUM_WORKERS // ROWS
B_PER_W = COLS // W_PER_ROW
CHUNK = 32
NBUF = 4
NCHUNKS = B_PER_W // CHUNK

_mesh = plsc.VectorSubcoreMesh(core_axis_name="c", subcore_axis_name="s")


@functools.partial(
    pl.kernel,
    mesh=_mesh,
    out_type=jax.ShapeDtypeStruct((ROWS, COLS, EMB_DIM), jnp.float32),
    scratch_types=[
        pltpu.VMEM((B_PER_W,), jnp.int32),
        pltpu.VMEM((NBUF, CHUNK, EMB_DIM), jnp.float32),
        pltpu.SemaphoreType.DMA,
        pltpu.SemaphoreType.DMA,
    ],
)
def _emb_lookup(table_hbm, idx_hbm, out_hbm, idx_v, rows_v, gsem, wsem):
    wid = lax.axis_index("s") * 2 + lax.axis_index("c")
    r = wid // W_PER_ROW
    c0 = (wid % W_PER_ROW) * B_PER_W
    pltpu.sync_copy(idx_hbm.at[r, pl.ds(c0, B_PER_W)], idx_v)
    def body(i, carry):
        b0 = 2 * i * CHUNK
        g0 = pltpu.async_copy(
            table_hbm.at[idx_v.at[pl.ds(b0, CHUNK)]], rows_v.at[0], gsem)
        g1 = pltpu.async_copy(
            table_hbm.at[idx_v.at[pl.ds(b0 + CHUNK, CHUNK)]],
            rows_v.at[1], gsem)
        g0.wait()
        w0 = pltpu.async_copy(
            rows_v.at[0], out_hbm.at[r, pl.ds(c0 + b0, CHUNK)], wsem)
        g1.wait()
        w1 = pltpu.async_copy(
            rows_v.at[1], out_hbm.at[r, pl.ds(c0 + b0 + CHUNK, CHUNK)], wsem)
        w0.wait()
        w1.wait()
        return carry

    lax.fori_loop(0, NCHUNKS // 2, body, 0)


def kernel(x, table):
    return _emb_lookup(table, x.astype(jnp.int32))


# speedup vs baseline: 1.0071x; 1.0071x over previous
import functools

import jax
import jax.numpy as jnp
from jax import lax
from jax.experimental import pallas as pl
from jax.experimental.pallas import tpu as pltpu
from jax.experimental.pallas import tpu_sc as plsc

ROWS = 4
COLS = 2048
EMB_DIM = 768
NUM_WORKERS = 32
W_PER_ROW = NUM_WORKERS // ROWS
B_PER_W = COLS // W_PER_ROW
CHUNK = 32
NBUF = 2
SP_NBUF = 3
NCHUNKS = B_PER_W // CHUNK

_mesh = plsc.VectorSubcoreMesh(core_axis_name="c", subcore_axis_name="s")


@functools.partial(
    pl.kernel,
    mesh=_mesh,
    out_type=jax.ShapeDtypeStruct((ROWS, COLS, EMB_DIM), jnp.float32),
    scratch_types=[
        pltpu.VMEM((B_PER_W,), jnp.int32),
        pltpu.VMEM((NBUF, CHUNK, EMB_DIM), jnp.float32),
        pltpu.VMEM_SHARED((16, SP_NBUF, CHUNK, EMB_DIM), jnp.float32),
        pltpu.SemaphoreType.DMA,
        pltpu.SemaphoreType.DMA,
        pltpu.SemaphoreType.DMA,
    ],
)
def _emb_lookup(table_hbm, idx_hbm, out_hbm, idx_v, rows_v, sp, gsem, xsem,
                wsem):
    sid = lax.axis_index("s")
    wid = sid * 2 + lax.axis_index("c")
    r = wid // W_PER_ROW
    c0 = (wid % W_PER_ROW) * B_PER_W
    pltpu.sync_copy(idx_hbm.at[r, pl.ds(c0, B_PER_W)], idx_v)
    g = [None] * NCHUNKS
    x = [None] * NCHUNKS
    w = [None] * NCHUNKS
    for ci in range(NBUF):
        g[ci] = pltpu.async_copy(
            table_hbm.at[idx_v.at[pl.ds(ci * CHUNK, CHUNK)]],
            rows_v.at[ci], gsem)
    for ci in range(NCHUNKS):
        if ci >= SP_NBUF:
            w[ci - SP_NBUF].wait()
        g[ci].wait()
        x[ci] = pltpu.async_copy(rows_v.at[ci % NBUF],
                                 sp.at[sid, ci % SP_NBUF], xsem)
        if ci >= 1:
            x[ci - 1].wait()
            w[ci - 1] = pltpu.async_copy(
                sp.at[sid, (ci - 1) % SP_NBUF],
                out_hbm.at[r, pl.ds(c0 + (ci - 1) * CHUNK, CHUNK)], wsem)
            nx = ci - 1 + NBUF
            if nx < NCHUNKS:
                g[nx] = pltpu.async_copy(
                    table_hbm.at[idx_v.at[pl.ds(nx * CHUNK, CHUNK)]],
                    rows_v.at[nx % NBUF], gsem)
    x[NCHUNKS - 1].wait()
    w[NCHUNKS - 1] = pltpu.async_copy(
        sp.at[sid, (NCHUNKS - 1) % SP_NBUF],
        out_hbm.at[r, pl.ds(c0 + (NCHUNKS - 1) * CHUNK, CHUNK)], wsem)
    for ci in range(max(0, NCHUNKS - NBUF), NCHUNKS):
        w[ci].wait()


def kernel(x, table):
    return _emb_lookup(table, x.astype(jnp.int32))


# speedup vs baseline: 1.0208x; 1.0136x over previous
import functools

import jax
import jax.numpy as jnp
from jax import lax
from jax.experimental import pallas as pl
from jax.experimental.pallas import tpu as pltpu
from jax.experimental.pallas import tpu_sc as plsc

ROWS = 4
COLS = 2048
EMB_DIM = 768
NUM_WORKERS = 32
W_PER_ROW = NUM_WORKERS // ROWS
B_PER_W = COLS // W_PER_ROW
CHUNK = 32
NBUF = 4
NCHUNKS = B_PER_W // CHUNK

_mesh = plsc.VectorSubcoreMesh(core_axis_name="c", subcore_axis_name="s")


@functools.partial(
    pl.kernel,
    mesh=_mesh,
    out_type=jax.ShapeDtypeStruct((ROWS, COLS, EMB_DIM), jnp.float32),
    scratch_types=[
        pltpu.VMEM((B_PER_W,), jnp.int32),
        pltpu.VMEM((NBUF, CHUNK, EMB_DIM), jnp.float32),
        pltpu.SemaphoreType.DMA,
        pltpu.SemaphoreType.DMA,
    ],
)
def _emb_lookup(table_hbm, idx_hbm, out_hbm, idx_v, rows_v, gsem, wsem):
    wid = lax.axis_index("s") * 2 + lax.axis_index("c")
    r = wid // W_PER_ROW
    c0 = (wid % W_PER_ROW) * B_PER_W
    pltpu.sync_copy(idx_hbm.at[r, pl.ds(c0, B_PER_W)], idx_v)
    g = [None] * NCHUNKS
    w = [None] * NCHUNKS
    for ci in range(NBUF):
        g[ci] = pltpu.async_copy(
            table_hbm.at[idx_v.at[pl.ds(ci * CHUNK, CHUNK)]],
            rows_v.at[ci], gsem)
    for ci in range(NCHUNKS):
        g[ci].wait()
        w[ci] = pltpu.async_copy(
            rows_v.at[ci % NBUF],
            out_hbm.at[r, pl.ds(c0 + ci * CHUNK, CHUNK)], wsem)
        nx = ci - 1 + NBUF
        if ci >= 1 and nx < NCHUNKS:
            w[ci - 1].wait()
            g[nx] = pltpu.async_copy(
                table_hbm.at[idx_v.at[pl.ds(nx * CHUNK, CHUNK)]],
                rows_v.at[nx % NBUF], gsem)
    for ci in range(max(0, NCHUNKS - NBUF), NCHUNKS):
        w[ci].wait()


def kernel(x, table):
    return _emb_lookup(table, x.astype(jnp.int32))
